# baseline (speedup 1.0000x reference)
import functools

import jax
import jax.numpy as jnp
from jax import lax
from jax.experimental import pallas as pl
from jax.experimental.pallas import tpu as pltpu

_GEM_EPS = 1e-6
_BN_EPS = 1e-5
_ONE_THIRD = 1.0 / 3.0


def _fused_kernel(xl_ref, xh_ref, gamma_ref, beta_ref, w_hbm_ref,
                  cls_ref, bn_ref, gf_ref, w_vmem, w_sem,
                  *, p_steps, tcl, tch, c_h, inv_hw_l, inv_hw_h):
    j = pl.program_id(0)

    @pl.when(j == 0)
    def _start_w_copy():
        pltpu.make_async_copy(w_hbm_ref, w_vmem, w_sem).start()

    @pl.when(j < p_steps)
    def _pool():
        xl = xl_ref[...]
        s1l = jnp.sum(xl, axis=1)
        xcl = jnp.maximum(xl, _GEM_EPS)
        s3l = jnp.sum(xcl * xcl * xcl, axis=1)
        geml = jnp.exp(jnp.log(s3l * inv_hw_l) * _ONE_THIRD)
        gf_ref[:, pl.ds(c_h + j * tcl, tcl)] = geml + s1l * inv_hw_l

        xh = xh_ref[...]
        s1h = jnp.sum(xh, axis=1)
        xch = jnp.maximum(xh, _GEM_EPS)
        s3h = jnp.sum(xch * xch * xch, axis=1)
        gemh = jnp.exp(jnp.log(s3h * inv_hw_h) * _ONE_THIRD)
        gf_ref[:, pl.ds(j * tch, tch)] = gemh + s1h * inv_hw_h

    @pl.when(j == p_steps)
    def _head():
        g = gf_ref[...]
        mean = jnp.mean(g, axis=0, keepdims=True)
        var = jnp.mean((g - mean) ** 2, axis=0, keepdims=True)
        y = (g - mean) * lax.rsqrt(var + _BN_EPS) * gamma_ref[...] + beta_ref[...]
        bn_ref[...] = y
        pltpu.make_async_copy(w_hbm_ref, w_vmem, w_sem).wait()
        cls_ref[...] = jnp.dot(y, w_vmem[...],
                               preferred_element_type=jnp.float32)


def _fused_forward(x_low, x_hi, gamma, beta, w_t, *, p_steps=8):
    n, hw_l, c_l = x_low.shape
    _, hw_h, c_h = x_hi.shape
    c = c_l + c_h
    k = w_t.shape[1]
    tcl = c_l // p_steps
    tch = c_h // p_steps
    steps = p_steps + 1
    last = p_steps - 1

    return pl.pallas_call(
        functools.partial(_fused_kernel, p_steps=p_steps, tcl=tcl, tch=tch,
                          c_h=c_h, inv_hw_l=1.0 / hw_l, inv_hw_h=1.0 / hw_h),
        out_shape=(
            jax.ShapeDtypeStruct((n, k), jnp.float32),
            jax.ShapeDtypeStruct((n, c), jnp.float32),
            jax.ShapeDtypeStruct((n, c), jnp.float32),
        ),
        grid=(steps,),
        in_specs=[
            pl.BlockSpec((n, hw_l, tcl), lambda j: (0, 0, jnp.minimum(j, last))),
            pl.BlockSpec((n, hw_h, tch), lambda j: (0, 0, jnp.minimum(j, last))),
            pl.BlockSpec((1, c), lambda j: (0, 0)),
            pl.BlockSpec((1, c), lambda j: (0, 0)),
            pl.BlockSpec(memory_space=pl.ANY),
        ],
        out_specs=(
            pl.BlockSpec((n, k), lambda j: (0, 0)),
            pl.BlockSpec((n, c), lambda j: (0, 0)),
            pl.BlockSpec((n, c), lambda j: (0, 0)),
        ),
        scratch_shapes=[
            pltpu.VMEM((c, k), jnp.float32),
            pltpu.SemaphoreType.DMA,
        ],
        compiler_params=pltpu.CompilerParams(
            dimension_semantics=("arbitrary",)),
    )(x_low, x_hi, gamma, beta, w_t)


def kernel(featmap_low, featmap, gamma, beta, w_t):
    n, c_l, h_l, w_l = featmap_low.shape
    _, c_h, h_h, w_h = featmap.shape
    x_low = jnp.transpose(featmap_low, (0, 2, 3, 1)).reshape(n, h_l * w_l, c_l)
    x_hi = jnp.transpose(featmap, (0, 2, 3, 1)).reshape(n, h_h * w_h, c_h)
    return _fused_forward(x_low, x_hi, gamma, beta, w_t)


# speedup vs baseline: 1.1099x; 1.1099x over previous
import functools

import jax
import jax.numpy as jnp
from jax import lax
from jax.experimental import pallas as pl
from jax.experimental.pallas import tpu as pltpu

_GEM_EPS = 1e-6
_BN_EPS = 1e-5
_ONE_THIRD = 1.0 / 3.0


def _fused_kernel(xl_ref, xh_ref, gamma_ref, beta_ref, w_hbm_ref,
                  cls_ref, bn_ref, gf_ref, w_vmem, w_sem,
                  *, p_steps, tcl, tch, c_h, inv_hw_l, inv_hw_h):
    j = pl.program_id(0)

    @pl.when(j == 0)
    def _start_w_copy():
        pltpu.make_async_copy(w_hbm_ref, w_vmem, w_sem).start()

    def _pool_sums(x, ch=8):
        hw = x.shape[1]
        a1 = x[:, 0:ch, :]
        cq = jnp.maximum(a1, _GEM_EPS)
        a3 = cq * cq * cq
        for q in range(1, hw // ch):
            xq = x[:, q * ch : (q + 1) * ch, :]
            cq = jnp.maximum(xq, _GEM_EPS)
            a1 = a1 + xq
            a3 = a3 + cq * cq * cq
        return jnp.sum(a1, axis=1), jnp.sum(a3, axis=1)

    @pl.when(j < p_steps)
    def _pool():
        s1l, s3l = _pool_sums(xl_ref[...])
        geml = jnp.exp(jnp.log(s3l * inv_hw_l) * _ONE_THIRD)
        gf_ref[:, pl.ds(c_h + j * tcl, tcl)] = geml + s1l * inv_hw_l

        s1h, s3h = _pool_sums(xh_ref[...])
        gemh = jnp.exp(jnp.log(s3h * inv_hw_h) * _ONE_THIRD)
        gf_ref[:, pl.ds(j * tch, tch)] = gemh + s1h * inv_hw_h

    @pl.when(j == p_steps)
    def _head():
        g = gf_ref[...]
        mean = jnp.mean(g, axis=0, keepdims=True)
        var = jnp.mean((g - mean) ** 2, axis=0, keepdims=True)
        y = (g - mean) * lax.rsqrt(var + _BN_EPS) * gamma_ref[...] + beta_ref[...]
        bn_ref[...] = y
        pltpu.make_async_copy(w_hbm_ref, w_vmem, w_sem).wait()
        cls_ref[...] = jnp.dot(y, w_vmem[...],
                               preferred_element_type=jnp.float32)


def _fused_forward(x_low, x_hi, gamma, beta, w_t, *, p_steps=8):
    n, hw_l, c_l = x_low.shape
    _, hw_h, c_h = x_hi.shape
    c = c_l + c_h
    k = w_t.shape[1]
    tcl = c_l // p_steps
    tch = c_h // p_steps
    steps = p_steps + 1
    last = p_steps - 1

    return pl.pallas_call(
        functools.partial(_fused_kernel, p_steps=p_steps, tcl=tcl, tch=tch,
                          c_h=c_h, inv_hw_l=1.0 / hw_l, inv_hw_h=1.0 / hw_h),
        out_shape=(
            jax.ShapeDtypeStruct((n, k), jnp.float32),
            jax.ShapeDtypeStruct((n, c), jnp.float32),
            jax.ShapeDtypeStruct((n, c), jnp.float32),
        ),
        grid=(steps,),
        in_specs=[
            pl.BlockSpec((n, hw_l, tcl), lambda j: (0, 0, jnp.minimum(j, last))),
            pl.BlockSpec((n, hw_h, tch), lambda j: (0, 0, jnp.minimum(j, last))),
            pl.BlockSpec((1, c), lambda j: (0, 0)),
            pl.BlockSpec((1, c), lambda j: (0, 0)),
            pl.BlockSpec(memory_space=pl.ANY),
        ],
        out_specs=(
            pl.BlockSpec((n, k), lambda j: (0, 0)),
            pl.BlockSpec((n, c), lambda j: (0, 0)),
            pl.BlockSpec((n, c), lambda j: (0, 0)),
        ),
        scratch_shapes=[
            pltpu.VMEM((c, k), jnp.float32),
            pltpu.SemaphoreType.DMA,
        ],
        compiler_params=pltpu.CompilerParams(
            dimension_semantics=("arbitrary",)),
    )(x_low, x_hi, gamma, beta, w_t)


def kernel(featmap_low, featmap, gamma, beta, w_t):
    n, c_l, h_l, w_l = featmap_low.shape
    _, c_h, h_h, w_h = featmap.shape
    x_low = jnp.transpose(featmap_low, (0, 2, 3, 1)).reshape(n, h_l * w_l, c_l)
    x_hi = jnp.transpose(featmap, (0, 2, 3, 1)).reshape(n, h_h * w_h, c_h)
    return _fused_forward(x_low, x_hi, gamma, beta, w_t)


# speedup vs baseline: 1.1459x; 1.0325x over previous
import functools

import jax
import jax.numpy as jnp
from jax import lax
from jax.experimental import pallas as pl
from jax.experimental.pallas import tpu as pltpu

_GEM_EPS = 1e-6
_BN_EPS = 1e-5
_ONE_THIRD = 1.0 / 3.0


def _fused_kernel(xl_ref, xh_ref, gamma_ref, beta_ref, w_hbm_ref,
                  cls_ref, bn_ref, gf_ref, w_vmem, w_sem,
                  *, p_steps, tcl, tch, c_h, inv_hw_l, inv_hw_h):
    j = pl.program_id(0)

    @pl.when(j == 0)
    def _start_w_copy():
        pltpu.make_async_copy(w_hbm_ref, w_vmem, w_sem).start()

    def _pool_sums(x, ch=8):
        hw = x.shape[1]
        a1 = x[:, 0:ch, :]
        cq = jnp.maximum(a1, _GEM_EPS)
        a3 = cq * cq * cq
        for q in range(1, hw // ch):
            xq = x[:, q * ch : (q + 1) * ch, :]
            cq = jnp.maximum(xq, _GEM_EPS)
            a1 = a1 + xq
            a3 = a3 + cq * cq * cq
        return jnp.sum(a1, axis=1), jnp.sum(a3, axis=1)

    @pl.when(j < p_steps)
    def _pool():
        s1l, s3l = _pool_sums(xl_ref[...])
        geml = jnp.exp(jnp.log(s3l * inv_hw_l) * _ONE_THIRD)
        gf_ref[:, pl.ds(c_h + j * tcl, tcl)] = geml + s1l * inv_hw_l

        s1h, s3h = _pool_sums(xh_ref[...])
        gemh = jnp.exp(jnp.log(s3h * inv_hw_h) * _ONE_THIRD)
        gf_ref[:, pl.ds(j * tch, tch)] = gemh + s1h * inv_hw_h

    @pl.when(j == p_steps)
    def _head():
        g = gf_ref[...]
        mean = jnp.mean(g, axis=0, keepdims=True)
        var = jnp.mean((g - mean) ** 2, axis=0, keepdims=True)
        y = (g - mean) * lax.rsqrt(var + _BN_EPS) * gamma_ref[...] + beta_ref[...]
        bn_ref[...] = y
        pltpu.make_async_copy(w_hbm_ref, w_vmem, w_sem).wait()
        cls_ref[...] = jnp.dot(y, w_vmem[...],
                               preferred_element_type=jnp.float32)


def _fused_forward(x_low, x_hi, gamma, beta, w_t, *, p_steps=4):
    n, hw_l, c_l = x_low.shape
    _, hw_h, c_h = x_hi.shape
    c = c_l + c_h
    k = w_t.shape[1]
    tcl = c_l // p_steps
    tch = c_h // p_steps
    steps = p_steps + 1
    last = p_steps - 1

    return pl.pallas_call(
        functools.partial(_fused_kernel, p_steps=p_steps, tcl=tcl, tch=tch,
                          c_h=c_h, inv_hw_l=1.0 / hw_l, inv_hw_h=1.0 / hw_h),
        out_shape=(
            jax.ShapeDtypeStruct((n, k), jnp.float32),
            jax.ShapeDtypeStruct((n, c), jnp.float32),
            jax.ShapeDtypeStruct((n, c), jnp.float32),
        ),
        grid=(steps,),
        in_specs=[
            pl.BlockSpec((n, hw_l, tcl), lambda j: (0, 0, jnp.minimum(j, last))),
            pl.BlockSpec((n, hw_h, tch), lambda j: (0, 0, jnp.minimum(j, last))),
            pl.BlockSpec((1, c), lambda j: (0, 0)),
            pl.BlockSpec((1, c), lambda j: (0, 0)),
            pl.BlockSpec(memory_space=pl.ANY),
        ],
        out_specs=(
            pl.BlockSpec((n, k), lambda j: (0, 0)),
            pl.BlockSpec((n, c), lambda j: (0, 0)),
            pl.BlockSpec((n, c), lambda j: (0, 0)),
        ),
        scratch_shapes=[
            pltpu.VMEM((c, k), jnp.float32),
            pltpu.SemaphoreType.DMA,
        ],
        compiler_params=pltpu.CompilerParams(
            dimension_semantics=("arbitrary",)),
    )(x_low, x_hi, gamma, beta, w_t)


def kernel(featmap_low, featmap, gamma, beta, w_t):
    n, c_l, h_l, w_l = featmap_low.shape
    _, c_h, h_h, w_h = featmap.shape
    x_low = jnp.transpose(featmap_low, (0, 2, 3, 1)).reshape(n, h_l * w_l, c_l)
    x_hi = jnp.transpose(featmap, (0, 2, 3, 1)).reshape(n, h_h * w_h, c_h)
    return _fused_forward(x_low, x_hi, gamma, beta, w_t)
